# baseline (speedup 1.0000x reference)
import functools

import jax
import jax.numpy as jnp
from jax import lax
from jax.experimental import pallas as pl
from jax.experimental.pallas import tpu as pltpu
from jax.experimental.pallas import tpu_sc as plsc

_B = 2
_C = 192
_ROW = 224 * 224
_NROWS = _B * _C

_info = plsc.get_sparse_core_info()
_NC = _info.num_cores
_NS = _info.num_subcores
_NW = _NC * _NS
_RPW = _NROWS // _NW


def _body(x_hbm, s_hbm, out_hbm, sem):
    wid = lax.axis_index("s") * _NC + lax.axis_index("c")
    base = wid * _RPW
    for i in range(_RPW):
        r = base + i
        is_x = jnp.logical_or(r == 0, r == _C)
        s_row = r + jnp.where(r >= _C, _C, 0)

        @pl.when(is_x)
        def _(r=r):
            pltpu.make_async_copy(x_hbm.at[r], out_hbm.at[r], sem).start()

        @pl.when(jnp.logical_not(is_x))
        def _(r=r, s_row=s_row):
            pltpu.make_async_copy(s_hbm.at[s_row], out_hbm.at[r], sem).start()
    for _i in range(_RPW):
        pltpu.make_async_copy(x_hbm.at[0], out_hbm.at[0], sem).wait()


def kernel(x, shortcut_input):
    xf = x.reshape(_NROWS, _ROW)
    sf = shortcut_input.reshape(_B * 2 * _C, _ROW)
    mesh = plsc.VectorSubcoreMesh(core_axis_name="c", subcore_axis_name="s")
    run = functools.partial(
        pl.kernel,
        mesh=mesh,
        out_type=jax.ShapeDtypeStruct((_NROWS, _ROW), jnp.float32),
        scratch_types=[pltpu.SemaphoreType.DMA],
    )(_body)
    out = run(xf, sf)
    return out.reshape(_B, _C, 224, 224)


# speedup vs baseline: 5.5518x; 5.5518x over previous
import functools

import jax
import jax.numpy as jnp
from jax import lax
from jax.experimental import pallas as pl
from jax.experimental.pallas import tpu as pltpu
from jax.experimental.pallas import tpu_sc as plsc

_B = 2
_C = 192
_ROW = 224 * 224
_SLAB = 191 * _ROW
_TOT = 2 * _SLAB

_NC = 2
_NS = 16
_NW = _NC * _NS
_Q = _TOT // _NW
_NCHUNK = 14
_CHUNK = _Q // _NCHUNK
_XPW = 2 * _ROW // _NW


def _body(x0_hbm, s_hbm, out_hbm, buf0, buf1, sem_l0, sem_l1, sem_s0, sem_s1):
    bufs = (buf0, buf1)
    wid = lax.axis_index("s") * _NC + lax.axis_index("c")
    base = wid * _Q
    sems_l = (sem_l0, sem_l1)
    sems_s = (sem_s0, sem_s1)

    def offsets(k):
        g = base + k * _CHUNK
        in_b1 = g >= _SLAB
        out_e = g + _ROW + jnp.where(in_b1, _ROW, 0)
        src_e = out_e + jnp.where(in_b1, _C * _ROW, 0)
        return src_e, out_e

    loads = {}
    stores = {}

    def start_load(k):
        src_e, _ = offsets(k)
        h = pltpu.make_async_copy(
            s_hbm.at[pl.ds(src_e, _CHUNK)], bufs[k % 2], sems_l[k % 2])
        h.start()
        loads[k] = h

    def start_store(k):
        _, out_e = offsets(k)
        h = pltpu.make_async_copy(
            bufs[k % 2], out_hbm.at[pl.ds(out_e, _CHUNK)], sems_s[k % 2])
        h.start()
        stores[k] = h

    start_load(0)
    for k in range(_NCHUNK):
        if k + 1 < _NCHUNK:
            if k - 1 >= 0:
                stores[k - 1].wait()
            start_load(k + 1)
        loads[k].wait()
        start_store(k)
    stores[_NCHUNK - 2].wait()
    stores[_NCHUNK - 1].wait()

    g2 = wid * _XPW
    out_e2 = g2 + jnp.where(g2 >= _ROW, (_C - 1) * _ROW, 0)
    h = pltpu.make_async_copy(
        x0_hbm.at[pl.ds(g2, _XPW)], buf0.at[pl.ds(0, _XPW)], sem_l0)
    h.start()
    h.wait()
    h = pltpu.make_async_copy(
        buf0.at[pl.ds(0, _XPW)], out_hbm.at[pl.ds(out_e2, _XPW)], sem_s0)
    h.start()
    h.wait()


def kernel(x, shortcut_input):
    x0f = x[:, 0, :, :].reshape(2 * _ROW)
    sf = shortcut_input.reshape(_B * 2 * _C * _ROW)
    mesh = plsc.VectorSubcoreMesh(
        core_axis_name="c", subcore_axis_name="s",
        num_cores=_NC, num_subcores=_NS)
    run = functools.partial(
        pl.kernel,
        mesh=mesh,
        out_type=jax.ShapeDtypeStruct((_B * _C * _ROW,), jnp.float32),
        scratch_types=[
            pltpu.VMEM((_CHUNK,), jnp.float32),
            pltpu.VMEM((_CHUNK,), jnp.float32),
            pltpu.SemaphoreType.DMA,
            pltpu.SemaphoreType.DMA,
            pltpu.SemaphoreType.DMA,
            pltpu.SemaphoreType.DMA,
        ],
    )(_body)
    out = run(x0f, sf)
    return out.reshape(_B, _C, 224, 224)


# speedup vs baseline: 11.9018x; 2.1438x over previous
import functools

import jax
import jax.numpy as jnp
from jax import lax
from jax.experimental import pallas as pl
from jax.experimental.pallas import tpu as pltpu
from jax.experimental.pallas import tpu_sc as plsc

_B = 2
_C = 192
_H = 224
_W = 224
_NIMG = _B * _C

_NC = 2
_NS = 16
_NW = _NC * _NS
_IPW = _NIMG // _NW


def _body(x_hbm, s_hbm, out_hbm, buf0, buf1, sem_l0, sem_l1, sem_s0, sem_s1):
    bufs = (buf0, buf1)
    sems_l = (sem_l0, sem_l1)
    sems_s = (sem_s0, sem_s1)
    wid = lax.axis_index("s") * _NC + lax.axis_index("c")
    base = wid * _IPW

    def coords(k):
        r = base + k
        b = jnp.where(r >= _C, 1, 0)
        c = r - b * _C
        return b, c

    def start_load(k):
        b, c = coords(k)
        is_x = c == 0

        @pl.when(is_x)
        def _():
            pltpu.make_async_copy(
                x_hbm.at[b, 0], bufs[k % 2], sems_l[k % 2]).start()

        @pl.when(jnp.logical_not(is_x))
        def _():
            pltpu.make_async_copy(
                s_hbm.at[b, c], bufs[k % 2], sems_l[k % 2]).start()

    def wait_load(k):
        pltpu.make_async_copy(
            s_hbm.at[0, 0], bufs[k % 2], sems_l[k % 2]).wait()

    stores = {}

    def start_store(k):
        b, c = coords(k)
        h = pltpu.make_async_copy(
            bufs[k % 2], out_hbm.at[b, c], sems_s[k % 2])
        h.start()
        stores[k] = h

    start_load(0)
    for k in range(_IPW):
        if k + 1 < _IPW:
            if k - 1 >= 0:
                stores[k - 1].wait()
            start_load(k + 1)
        wait_load(k)
        start_store(k)
    stores[_IPW - 2].wait()
    stores[_IPW - 1].wait()


def kernel(x, shortcut_input):
    mesh = plsc.VectorSubcoreMesh(
        core_axis_name="c", subcore_axis_name="s",
        num_cores=_NC, num_subcores=_NS)
    run = functools.partial(
        pl.kernel,
        mesh=mesh,
        out_type=jax.ShapeDtypeStruct((_B, _C, _H, _W), jnp.float32),
        scratch_types=[
            pltpu.VMEM((_H, _W), jnp.float32),
            pltpu.VMEM((_H, _W), jnp.float32),
            pltpu.SemaphoreType.DMA,
            pltpu.SemaphoreType.DMA,
            pltpu.SemaphoreType.DMA,
            pltpu.SemaphoreType.DMA,
        ],
        compiler_params=pltpu.CompilerParams(use_tc_tiling_on_sc=True),
    )(_body)
    return run(x, shortcut_input)
